# Initial kernel scaffold; baseline (speedup 1.0000x reference)
import jax
import jax.numpy as jnp
from jax.experimental import pallas as pl


def kernel(times_in, times_out, segment_filter_ids, one_hot_predecessor_ids, decay_rate, kernel, bias):
    raise NotImplementedError("write your pallas kernel here")


# speedup vs baseline: 52.4717x; 52.4717x over previous
import functools

import jax
import jax.numpy as jnp
from jax import lax
from jax.experimental import pallas as pl
from jax.experimental.pallas import tpu as pltpu
from jax.experimental.pallas import tpu_sc as plsc

N_IN = 262144
N_OUT = 262144
K = 5
F_IN = 4
F_OUT = 32
KC = K * F_IN

N_TPAD = N_IN + 1024
TOTAL = N_OUT * KC
NC, NS = 2, 16
NW = NC * NS
TOT_W = TOTAL // NW
CHUNK = 8192
GSUB = 128
N_CHUNKS = TOT_W // CHUNK


def _table_body(dr_ref, t_ref, s_ref, out_ref):
    t = t_ref[...]
    s = s_ref[...]
    arg = jnp.zeros_like(t)
    for c in range(F_IN):
        arg += jnp.where(s == c, dr_ref[c], 0.0)
    w = jnp.exp(arg * t)
    bits = lax.bitcast_convert_type(w, jnp.int32)
    packed = (bits & ~3) | (s & 3)
    out_ref[...] = jnp.where(s >= 0, packed, 0)


def _build_table(dr, t_pad, s_pad):
    rows = N_TPAD // 128
    return pl.pallas_call(
        _table_body,
        out_shape=jax.ShapeDtypeStruct((rows, 128), jnp.int32),
        in_specs=[
            pl.BlockSpec(memory_space=pltpu.MemorySpace.SMEM),
            pl.BlockSpec((rows, 128), lambda: (0, 0)),
            pl.BlockSpec((rows, 128), lambda: (0, 0)),
        ],
        out_specs=pl.BlockSpec((rows, 128), lambda: (0, 0)),
    )(dr, t_pad.reshape(rows, 128), s_pad.reshape(rows, 128))


def _gather_body(table_hbm, ids_hbm, out_hbm, idx_v, rows_v, sem):
    wid = lax.axis_index("s") * NC + lax.axis_index("c")
    base = wid * TOT_W

    def chunk_body(ci, carry):
        off = base + ci * CHUNK
        pltpu.sync_copy(ids_hbm.at[pl.ds(off, CHUNK)], idx_v)

        def fire(m, carry2):
            pltpu.async_copy(
                table_hbm.at[idx_v.at[pl.ds(m * GSUB, GSUB)]],
                rows_v.at[pl.ds(m * GSUB, GSUB)],
                sem,
            )
            return carry2

        lax.fori_loop(0, CHUNK // GSUB, fire, 0)
        pltpu.make_async_copy(table_hbm.at[pl.ds(0, CHUNK)], rows_v, sem).wait()
        pltpu.sync_copy(rows_v, out_hbm.at[pl.ds(off, CHUNK)])
        return carry

    lax.fori_loop(0, N_CHUNKS, chunk_body, 0)


@functools.lru_cache(maxsize=None)
def _make_gather():
    return functools.partial(
        pl.kernel,
        mesh=plsc.VectorSubcoreMesh(
            core_axis_name="c", subcore_axis_name="s",
            num_cores=NC, num_subcores=NS,
        ),
        out_type=jax.ShapeDtypeStruct((TOTAL,), jnp.int32),
        scratch_types=[
            pltpu.VMEM((CHUNK,), jnp.int32),
            pltpu.VMEM((CHUNK,), jnp.int32),
            pltpu.SemaphoreType.DMA,
        ],
    )(_gather_body)


def _sc_gather(table, ids_flat):
    return _make_gather()(table, ids_flat)


_ROWS_C = 512


def _out_body(g_ref, tout_ref, dr_ref, kern_ref, bias_ref, out_ref):
    bits = g_ref[...]
    w = lax.bitcast_convert_type(bits & ~3, jnp.float32)
    ch = bits & 3
    cpat = lax.broadcasted_iota(jnp.int32, (_ROWS_C, KC), 1) % F_IN
    mask = ch == cpat
    tout = tout_ref[...]
    e4 = jnp.exp(-tout * dr_ref[...])
    e20 = jnp.concatenate([e4] * K, axis=1)
    vals = jnp.where(mask, w * e20, 0.0)
    out_ref[...] = (
        jnp.dot(vals, kern_ref[...], preferred_element_type=jnp.float32)
        + bias_ref[...]
    )


def _finish(g, tout, dr_row, kern2d, bias_row):
    grid = (N_OUT // _ROWS_C,)
    return pl.pallas_call(
        _out_body,
        grid=grid,
        out_shape=jax.ShapeDtypeStruct((N_OUT, F_OUT), jnp.float32),
        in_specs=[
            pl.BlockSpec((_ROWS_C, KC), lambda i: (i, 0)),
            pl.BlockSpec((_ROWS_C, 1), lambda i: (i, 0)),
            pl.BlockSpec((1, F_IN), lambda i: (0, 0)),
            pl.BlockSpec((KC, F_OUT), lambda i: (0, 0)),
            pl.BlockSpec((1, F_OUT), lambda i: (0, 0)),
        ],
        out_specs=pl.BlockSpec((_ROWS_C, F_OUT), lambda i: (i, 0)),
    )(g, tout, dr_row, kern2d, bias_row)


def kernel(times_in, times_out, segment_filter_ids, one_hot_predecessor_ids,
           decay_rate, kernel, bias):
    dr = jax.nn.softplus(decay_rate)

    pad = N_TPAD - N_IN
    t_pad = jnp.pad(times_in, (0, pad))
    s_pad = jnp.pad(segment_filter_ids, (0, pad), constant_values=-1)

    table = _build_table(dr, t_pad, s_pad).reshape(-1)
    ids_flat = one_hot_predecessor_ids.reshape(-1)
    g = _sc_gather(table, ids_flat)

    out = _finish(
        g.reshape(N_OUT, KC),
        times_out.reshape(N_OUT, 1),
        dr.reshape(1, F_IN),
        kernel.reshape(KC, F_OUT),
        bias.reshape(1, F_OUT),
    )
    return out


# speedup vs baseline: 339.7322x; 6.4746x over previous
import functools

import jax
import jax.numpy as jnp
from jax import lax
from jax.experimental import pallas as pl
from jax.experimental.pallas import tpu as pltpu
from jax.experimental.pallas import tpu_sc as plsc

N_IN = 262144
N_OUT = 262144
K = 5
F_IN = 4
F_OUT = 32
KC = K * F_IN

N_TPAD = N_IN + 1024
TOTAL = N_OUT * KC
NC, NS = 2, 16
NW = NC * NS
TOT_W = TOTAL // NW
CHUNK = 8192
GSUB = 128
N_CHUNKS = TOT_W // CHUNK


def _table_body(dr_ref, t_ref, s_ref, out_ref):
    t = t_ref[...]
    s = s_ref[...]
    arg = jnp.zeros_like(t)
    for c in range(F_IN):
        arg += jnp.where(s == c, dr_ref[c], 0.0)
    w = jnp.exp(arg * t)
    bits = lax.bitcast_convert_type(w, jnp.int32)
    packed = (bits & ~3) | (s & 3)
    out_ref[...] = jnp.where(s >= 0, packed, 0)


def _build_table(dr, t_pad, s_pad):
    rows = N_TPAD // 128
    return pl.pallas_call(
        _table_body,
        out_shape=jax.ShapeDtypeStruct((rows, 128), jnp.int32),
        in_specs=[
            pl.BlockSpec(memory_space=pltpu.MemorySpace.SMEM),
            pl.BlockSpec((rows, 128), lambda: (0, 0)),
            pl.BlockSpec((rows, 128), lambda: (0, 0)),
        ],
        out_specs=pl.BlockSpec((rows, 128), lambda: (0, 0)),
    )(dr, t_pad.reshape(rows, 128), s_pad.reshape(rows, 128))


def _gather_body(table_hbm, ids_hbm, out_hbm, idx_v, rows_v, sem):
    wid = lax.axis_index("s") * NC + lax.axis_index("c")
    base = wid * TOT_W

    def chunk_body(ci, carry):
        off = base + ci * CHUNK
        pltpu.sync_copy(ids_hbm.at[pl.ds(off, CHUNK)], idx_v)

        def fire(m, carry2):
            pltpu.async_copy(
                table_hbm.at[idx_v.at[pl.ds(m * GSUB, GSUB)]],
                rows_v.at[pl.ds(m * GSUB, GSUB)],
                sem,
            )
            return carry2

        lax.fori_loop(0, CHUNK // GSUB, fire, 0)
        pltpu.make_async_copy(table_hbm.at[pl.ds(0, CHUNK)], rows_v, sem).wait()
        pltpu.sync_copy(rows_v, out_hbm.at[pl.ds(off, CHUNK)])
        return carry

    lax.fori_loop(0, N_CHUNKS, chunk_body, 0)


@functools.lru_cache(maxsize=None)
def _make_gather():
    return functools.partial(
        pl.kernel,
        mesh=plsc.VectorSubcoreMesh(
            core_axis_name="c", subcore_axis_name="s",
            num_cores=NC, num_subcores=NS,
        ),
        out_type=jax.ShapeDtypeStruct((TOTAL,), jnp.int32),
        scratch_types=[
            pltpu.VMEM((CHUNK,), jnp.int32),
            pltpu.VMEM((CHUNK,), jnp.int32),
            pltpu.SemaphoreType.DMA,
        ],
    )(_gather_body)


def _sc_gather(table, ids_flat):
    return _make_gather()(table, ids_flat)


_JB = 8
_NB = N_OUT // 128


def _out_body(dr_ref, g_ref, t_ref, m2_ref, b_ref, o_ref):
    bits = g_ref[...]
    w = lax.bitcast_convert_type(bits & ~3, jnp.float32)
    ch = bits & 3
    cpat = lax.broadcasted_iota(jnp.int32, (KC, _JB, 128), 0) % F_IN
    mask = ch == cpat
    t = t_ref[...]
    drsel = jnp.zeros((KC, _JB, 128), jnp.float32)
    for c in range(F_IN):
        drsel += jnp.where(cpat == c, dr_ref[c], 0.0)
    e = jnp.exp(-drsel * t[None, :, :])
    vals = jnp.where(mask, w * e, 0.0)
    m2 = m2_ref[...]
    b = b_ref[...]
    for s in range(_JB):
        o_ref[:, s * 128:(s + 1) * 128] = (
            lax.dot_general(m2, vals[:, s, :], (((0,), (0,)), ((), ())),
                            preferred_element_type=jnp.float32)
            + b
        )


def _finish(dr, g3, tout2, kern2d, bias2):
    grid = (_NB // _JB,)
    return pl.pallas_call(
        _out_body,
        grid=grid,
        out_shape=jax.ShapeDtypeStruct((F_OUT, N_OUT), jnp.float32),
        in_specs=[
            pl.BlockSpec(memory_space=pltpu.MemorySpace.SMEM),
            pl.BlockSpec((KC, _JB, 128), lambda i: (0, i, 0)),
            pl.BlockSpec((_JB, 128), lambda i: (i, 0)),
            pl.BlockSpec((KC, F_OUT), lambda i: (0, 0)),
            pl.BlockSpec((F_OUT, 1), lambda i: (0, 0)),
        ],
        out_specs=pl.BlockSpec((F_OUT, _JB * 128), lambda i: (0, i)),
    )(dr, g3, tout2, kern2d, bias2)


def kernel(times_in, times_out, segment_filter_ids, one_hot_predecessor_ids,
           decay_rate, kernel, bias):
    dr = jax.nn.softplus(decay_rate)

    pad = N_TPAD - N_IN
    t_pad = jnp.pad(times_in, (0, pad))
    s_pad = jnp.pad(segment_filter_ids, (0, pad), constant_values=-1)

    table = _build_table(dr, t_pad, s_pad).reshape(-1)
    ids_flat = jnp.transpose(one_hot_predecessor_ids, (1, 2, 0)).reshape(-1)
    g = _sc_gather(table, ids_flat)

    out_t = _finish(
        dr,
        g.reshape(KC, _NB, 128),
        times_out.reshape(_NB, 128),
        kernel.reshape(KC, F_OUT),
        bias.reshape(F_OUT, 1),
    )
    return out_t.T
